# baseline (speedup 1.0000x reference)
import functools

import jax
import jax.numpy as jnp
from jax import lax
from jax.experimental import pallas as pl
from jax.experimental.pallas import tpu as pltpu
from jax.experimental.pallas import tpu_sc as plsc

_NW = 32
_CL = 128
_BS = 128
_L = 16


def _sc_gather(n_chunks, s_chunks, d):
    assert n_chunks % 2 == 0
    tot = _NW * n_chunks * _CL
    stot = _NW * s_chunks * _CL
    hc = _CL // 2
    mesh = plsc.VectorSubcoreMesh(core_axis_name="c", subcore_axis_name="s")

    @functools.partial(
        pl.kernel,
        mesh=mesh,
        out_type=(
            jax.ShapeDtypeStruct((tot // 2, d), jnp.int32),
            jax.ShapeDtypeStruct((stot, d), jnp.float32),
        ),
        scratch_types=[
            pltpu.VMEM((n_chunks + 1, _CL), jnp.int32),
            pltpu.VMEM((s_chunks, _CL), jnp.int32),
            pltpu.VMEM((_CL, d), jnp.float32),
            pltpu.VMEM((_CL, d), jnp.float32),
            pltpu.VMEM((hc, d), jnp.int32),
            pltpu.VMEM((hc, d), jnp.int32),
            pltpu.VMEM((_CL, d), jnp.float32),
            pltpu.SemaphoreType.DMA,
            pltpu.SemaphoreType.DMA,
            pltpu.SemaphoreType.DMA,
            pltpu.SemaphoreType.DMA,
        ],
        compiler_params=pltpu.CompilerParams(needs_layout_passes=False),
    )
    def gather(table, idx, sidx, out, sout, idx_v, sidx_v, f0, f1, p0, p1,
               sbuf, ga0, ga1, st0, st1):
        cid = lax.axis_index("c")
        sid = lax.axis_index("s")
        wid = sid * 2 + cid
        base = wid * (n_chunks * hc)
        sbase = wid * (s_chunks * _CL)
        pltpu.sync_copy(idx.at[wid], idx_v)
        pltpu.sync_copy(sidx.at[wid], sidx_v)

        def ga(c, buf, sem):
            return pltpu.make_async_copy(table.at[idx_v.at[c]], buf, sem)

        def st(c, buf, sem):
            return pltpu.make_async_copy(
                buf, out.at[pl.ds(base + c * hc, hc)], sem)

        def pack(fbuf, pbuf):
            def prow(p, carry):
                for g in range(d // _L):
                    a = fbuf[2 * p, pl.ds(_L * g, _L)]
                    b = fbuf[2 * p + 1, pl.ds(_L * g, _L)]
                    w = plsc.bitcast(
                        plsc.pack(a, b, format=plsc.PackFormat.INTERLEAVED),
                        jnp.int32)
                    pbuf[p, pl.ds(_L * g, _L)] = w
                return carry
            lax.fori_loop(0, hc, prow, 0)

        ga(0, f0, ga0).start()
        ga(0, f0, ga0).wait()
        ga(1, f1, ga1).start()
        pack(f0, p0)
        st(0, p0, st0).start()
        ga(1, f1, ga1).wait()
        ga(2, f0, ga0).start()
        pack(f1, p1)
        st(1, p1, st1).start()

        def body(q, carry):
            c = 2 * q
            ga(c, f0, ga0).wait()
            ga(c + 1, f1, ga1).start()
            st(c - 2, p0, st0).wait()
            pack(f0, p0)
            st(c, p0, st0).start()
            ga(c + 1, f1, ga1).wait()
            ga(c + 2, f0, ga0).start()
            st(c - 1, p1, st1).wait()
            pack(f1, p1)
            st(c + 1, p1, st1).start()
            return carry

        lax.fori_loop(1, n_chunks // 2, body, 0)
        ga(n_chunks, f0, ga0).wait()
        st(n_chunks - 2, p0, st0).wait()
        st(n_chunks - 1, p1, st1).wait()

        def sbody(c, carry):
            pltpu.async_copy(table.at[sidx_v.at[c]], sbuf, ga0).wait()
            pltpu.sync_copy(sbuf, sout.at[pl.ds(sbase + c * _CL, _CL)])
            return carry

        lax.fori_loop(0, s_chunks, sbody, 0)

    return gather


def _tc_body(k, eu_ref, self_ref, w1e_ref, w1s_ref, b1_ref, w2_ref, b2_ref,
             w3_ref, out_ref):
    bs = _BS
    d = self_ref.shape[1]
    kh = k // 2
    x = eu_ref[...]
    e_a = lax.bitcast_convert_type(x << 16, jnp.float32)
    e_b = lax.bitcast_convert_type(x & jnp.int32(-65536), jnp.float32)
    s_f = self_ref[...]
    u1 = jnp.dot(s_f, w1s_ref[...], preferred_element_type=jnp.float32)
    u1r = jnp.broadcast_to(u1[:, None, :], (bs, kh, d)).reshape(bs * kh, d)

    def scores(e_u):
        h = jnp.dot(e_u, w1e_ref[...], preferred_element_type=jnp.float32)
        h = jnp.maximum(h + u1r + b1_ref[...], 0.0)
        h = jnp.dot(h, w2_ref[...], preferred_element_type=jnp.float32)
        h = jnp.maximum(h + b2_ref[...], 0.0)
        s = jnp.sum(h * w3_ref[...], axis=1, keepdims=True)
        return s.reshape(bs, kh, 1)

    s_a = scores(e_a)
    s_b = scores(e_b)
    m = jnp.maximum(jnp.max(s_a, axis=1, keepdims=True),
                    jnp.max(s_b, axis=1, keepdims=True))
    x_a = jnp.exp(s_a - m)
    x_b = jnp.exp(s_b - m)
    den = jnp.sum(x_a, axis=1, keepdims=True) + jnp.sum(
        x_b, axis=1, keepdims=True)
    w_a = x_a / den
    w_b = x_b / den
    att = (jnp.sum(w_a * e_a.reshape(bs, kh, d), axis=1) +
           jnp.sum(w_b * e_b.reshape(bs, kh, d), axis=1))
    out_ref[...] = (att + s_f) * 0.5


def _tc_mlp(bp, k, d):
    grid = bp // _BS
    body = functools.partial(_tc_body, k)
    pk_rows = _BS * k // 2
    return pl.pallas_call(
        body,
        grid=(grid,),
        in_specs=[
            pl.BlockSpec((pk_rows, d), lambda i: (i, 0)),
            pl.BlockSpec((_BS, d), lambda i: (i, 0)),
            pl.BlockSpec((d, d), lambda i: (0, 0)),
            pl.BlockSpec((d, d), lambda i: (0, 0)),
            pl.BlockSpec((1, d), lambda i: (0, 0)),
            pl.BlockSpec((d, d), lambda i: (0, 0)),
            pl.BlockSpec((1, d), lambda i: (0, 0)),
            pl.BlockSpec((1, d), lambda i: (0, 0)),
        ],
        out_specs=pl.BlockSpec((_BS, d), lambda i: (i, 0)),
        out_shape=jax.ShapeDtypeStruct((bp, d), jnp.float32),
    )


def kernel(nodes, to_neighs, u2e_weight, att1_W, att1_b, att2_W, att2_b,
           att3_W, att3_b):
    b = nodes.shape[0]
    k = to_neighs.shape[1]
    d = u2e_weight.shape[1]
    bp = ((b + _BS - 1) // _BS) * _BS
    eu_rows = bp * k
    chunk = _NW * _CL
    n_chunks = (eu_rows + chunk - 1) // chunk
    n_chunks += n_chunks % 2
    eu_tot = n_chunks * chunk
    s_chunks = (bp + chunk - 1) // chunk
    s_tot = s_chunks * chunk

    neigh = to_neighs[:, :, 0].astype(jnp.int32).reshape(b * k)
    nid = nodes.astype(jnp.int32)
    idx = jnp.concatenate([
        neigh, jnp.zeros((eu_tot - b * k,), jnp.int32),
    ]).reshape(_NW, n_chunks, _CL)
    idx = jnp.pad(idx, ((0, 0), (0, 1), (0, 0)))
    sidx = jnp.concatenate([
        nid, jnp.zeros((s_tot - b,), jnp.int32),
    ]).reshape(_NW, s_chunks, _CL)

    rows, selfr = _sc_gather(n_chunks, s_chunks, d)(u2e_weight, idx, sidx)

    w1e = att1_W[:d]
    w1s = att1_W[d:]
    out = _tc_mlp(bp, k, d)(
        rows, selfr, w1e, w1s, att1_b.reshape(1, d), att2_W,
        att2_b.reshape(1, d), att3_W.reshape(1, d))
    return out[:b]


# speedup vs baseline: 1.5506x; 1.5506x over previous
import functools

import jax
import jax.numpy as jnp
from jax import lax
from jax.experimental import pallas as pl
from jax.experimental.pallas import tpu as pltpu
from jax.experimental.pallas import tpu_sc as plsc

_NW = 32
_CL = 128
_BS = 128
def _sc_gather(n_chunks, d):
    assert n_chunks % 2 == 0
    tot = _NW * n_chunks * _CL
    mesh = plsc.VectorSubcoreMesh(core_axis_name="c", subcore_axis_name="s")

    @functools.partial(
        pl.kernel,
        mesh=mesh,
        out_type=jax.ShapeDtypeStruct((tot, d), jnp.float32),
        scratch_types=[
            pltpu.VMEM((n_chunks + 1, _CL), jnp.int32),
            pltpu.VMEM((_CL, d), jnp.float32),
            pltpu.VMEM((_CL, d), jnp.float32),
            pltpu.SemaphoreType.DMA,
            pltpu.SemaphoreType.DMA,
        ],
    )
    def gather(table, idx, out, idx_v, buf0, buf1, g0, g1):
        cid = lax.axis_index("c")
        sid = lax.axis_index("s")
        wid = sid * 2 + cid
        base = wid * (n_chunks * _CL)
        pltpu.sync_copy(idx.at[wid], idx_v)

        def chunk_in(c, buf, sem):
            pltpu.async_copy(table.at[idx_v.at[c]], buf, sem).wait()

        def store(c, buf, sem):
            return pltpu.make_async_copy(
                buf, out.at[pl.ds(base + c * _CL, _CL)], sem)

        chunk_in(0, buf0, g0)
        store(0, buf0, g0).start()
        chunk_in(1, buf1, g1)
        store(1, buf1, g1).start()

        def body(q, carry):
            c = 2 * q
            store(c - 2, buf0, g0).wait()
            chunk_in(c, buf0, g0)
            store(c, buf0, g0).start()
            store(c - 1, buf1, g1).wait()
            chunk_in(c + 1, buf1, g1)
            store(c + 1, buf1, g1).start()
            return carry

        lax.fori_loop(1, n_chunks // 2, body, 0)
        store(n_chunks - 2, buf0, g0).wait()
        store(n_chunks - 1, buf1, g1).wait()

    return gather


def _tc_body(k, eu_ref, self_ref, w1e_ref, w1s_ref, b1_ref, w2_ref, b2_ref,
             w3_ref, out_ref):
    bs = _BS
    d = self_ref.shape[1]
    e_u = eu_ref[...]
    s_f = self_ref[...]
    u1 = jnp.dot(s_f, w1s_ref[...], preferred_element_type=jnp.float32)
    u1r = jnp.broadcast_to(u1[:, None, :], (bs, k, d)).reshape(bs * k, d)
    h = jnp.dot(e_u, w1e_ref[...], preferred_element_type=jnp.float32)
    h = jnp.maximum(h + u1r + b1_ref[...], 0.0)
    h = jnp.dot(h, w2_ref[...], preferred_element_type=jnp.float32)
    h = jnp.maximum(h + b2_ref[...], 0.0)
    s = jnp.sum(h * w3_ref[...], axis=1, keepdims=True)
    s3 = s.reshape(bs, k, 1)
    m = jnp.max(s3, axis=1, keepdims=True)
    e = jnp.exp(s3 - m)
    w = e / jnp.sum(e, axis=1, keepdims=True)
    att = jnp.sum(w * e_u.reshape(bs, k, d), axis=1)
    out_ref[...] = (att + s_f) * 0.5


def _tc_mlp(bp, k, d, eu_rows):
    grid = bp // _BS
    body = functools.partial(_tc_body, k)
    return pl.pallas_call(
        body,
        grid=(grid,),
        in_specs=[
            pl.BlockSpec((_BS * k, d), lambda i: (i, 0)),
            pl.BlockSpec((_BS, d), lambda i: (eu_rows // _BS + i, 0)),
            pl.BlockSpec((d, d), lambda i: (0, 0)),
            pl.BlockSpec((d, d), lambda i: (0, 0)),
            pl.BlockSpec((1, d), lambda i: (0, 0)),
            pl.BlockSpec((d, d), lambda i: (0, 0)),
            pl.BlockSpec((1, d), lambda i: (0, 0)),
            pl.BlockSpec((1, d), lambda i: (0, 0)),
        ],
        out_specs=pl.BlockSpec((_BS, d), lambda i: (i, 0)),
        out_shape=jax.ShapeDtypeStruct((bp, d), jnp.float32),
    )


def kernel(nodes, to_neighs, u2e_weight, att1_W, att1_b, att2_W, att2_b,
           att3_W, att3_b):
    b = nodes.shape[0]
    k = to_neighs.shape[1]
    d = u2e_weight.shape[1]
    bp = ((b + _BS - 1) // _BS) * _BS
    eu_rows = bp * k
    raw = eu_rows + bp
    chunk = _NW * _CL
    n_chunks = (raw + chunk - 1) // chunk
    n_chunks += n_chunks % 2
    tot = n_chunks * chunk

    neigh = to_neighs[:, :, 0].astype(jnp.int32).reshape(b * k)
    nid = nodes.astype(jnp.int32)
    idx = jnp.concatenate([
        neigh,
        jnp.zeros((eu_rows - b * k,), jnp.int32),
        nid,
        jnp.zeros((tot - eu_rows - b,), jnp.int32),
    ]).reshape(_NW, n_chunks, _CL)
    idx = jnp.pad(idx, ((0, 0), (0, 1), (0, 0)))

    rows = _sc_gather(n_chunks, d)(u2e_weight, idx)

    w1e = att1_W[:d]
    w1s = att1_W[d:]
    out = _tc_mlp(bp, k, d, eu_rows)(
        rows, rows, w1e, w1s, att1_b.reshape(1, d), att2_W,
        att2_b.reshape(1, d), att3_W.reshape(1, d))
    return out[:b]
